# baseline (speedup 1.0000x reference)
import functools

import jax
import jax.numpy as jnp
from jax import lax
from jax.experimental import pallas as pl
from jax.experimental.pallas import tpu as pltpu
from jax.experimental.pallas import tpu_sc as plsc

N = 10000
E = 320000
M = 10000
C = 128
D = 144
NC = 2
NS = 16
NW = NC * NS
EPW = E // NW
K = 125
NCHUNK = EPW // K
RPS = M // NS


def _sc_body(t_hbm, eref_hbm, eq_hbm, z_hbm, out_hbm, idx_v, rows_v, acc_sh,
             sem, isem):
    c = lax.axis_index("c")
    s = lax.axis_index("s")
    wid = c * NS + s
    pltpu.sync_copy(z_hbm, acc_sh.at[pl.ds(s * RPS, RPS)])
    plsc.subcore_barrier()

    def fetch_idx(i, slot, do_async):
        if do_async:
            pltpu.async_copy(eref_hbm.at[wid, i], idx_v.at[slot, 0], isem)
            pltpu.async_copy(eq_hbm.at[wid, i], idx_v.at[slot, 1], isem)
        else:
            pltpu.sync_copy(eref_hbm.at[wid, i], idx_v.at[slot, 0])
            pltpu.sync_copy(eq_hbm.at[wid, i], idx_v.at[slot, 1])

    def wait_idx(i, slot):
        pltpu.make_async_copy(eref_hbm.at[wid, i], idx_v.at[slot, 0],
                              isem).wait()
        pltpu.make_async_copy(eq_hbm.at[wid, i], idx_v.at[slot, 1],
                              isem).wait()

    fetch_idx(0, 0, False)
    fetch_idx(1, 1, True)
    pltpu.async_copy(t_hbm.at[idx_v.at[0, 0]], rows_v.at[0], sem)

    def chunk(i, carry):
        b3 = lax.rem(i, 3)
        b2 = lax.rem(i, 2)
        n3 = lax.rem(i + 1, 3)

        @pl.when(i + 1 < NCHUNK)
        def _fire_gather():
            wait_idx(i + 1, n3)
            pltpu.async_copy(t_hbm.at[idx_v.at[n3, 0]], rows_v.at[1 - b2],
                             sem)

        @pl.when(i + 2 < NCHUNK)
        def _fire_idx():
            fetch_idx(i + 2, lax.rem(i + 2, 3), True)

        pltpu.make_async_copy(t_hbm.at[idx_v.at[b3, 0]], rows_v.at[b2],
                              sem).wait()
        pltpu.sync_copy(rows_v.at[b2], acc_sh.at[idx_v.at[b3, 1]], add=True)
        return carry

    lax.fori_loop(0, NCHUNK, chunk, 0)
    plsc.subcore_barrier()
    pltpu.sync_copy(acc_sh.at[pl.ds(s * RPS, RPS)],
                    out_hbm.at[pl.ds(s * RPS, RPS), c])


def _sc_scatter(table, eref_w, eq_w, zrows):
    mesh = plsc.VectorSubcoreMesh(
        core_axis_name="c", subcore_axis_name="s", num_cores=NC, num_subcores=NS)
    return pl.kernel(
        _sc_body,
        out_type=jax.ShapeDtypeStruct((M, NC, D), jnp.float32),
        mesh=mesh,
        scratch_types=[
            pltpu.VMEM((3, 2, K), jnp.int32),
            pltpu.VMEM((2, K, D), jnp.float32),
            pltpu.VMEM_SHARED((M, D), jnp.float32),
            pltpu.SemaphoreType.DMA,
            pltpu.SemaphoreType.DMA,
        ],
        compiler_params=pltpu.CompilerParams(use_tc_tiling_on_sc=False),
    )(table, eref_w, eq_w, zrows)


def _dense_body(x, qb, wm, wp, bias, gamma, beta, out):
    f32 = jnp.float32
    dg = functools.partial(lax.dot_general, preferred_element_type=f32)
    contract1 = (((1,), (1,)), ((), ()))
    wcomb = jnp.concatenate(
        [wm[...], wp[...], bias[...], jnp.zeros((C, D - C - 4), f32)], axis=1)
    w2 = jnp.concatenate([wcomb, wcomb], axis=1)
    col = lax.broadcasted_iota(jnp.int32, (C, 2 * D), 1)
    sel2 = jnp.where((col == C + 3) | (col == D + C + 3), 1.0, 0.0)
    wq = jnp.concatenate([jnp.zeros((C, 1), f32), wp[...]], axis=1)

    counts = dg(x[...], sel2, contract1)
    numer = dg(x[...], w2, contract1) - counts * dg(qb[...], wq, contract1)
    qf = numer / jnp.maximum(counts, 1.0)
    mean = jnp.mean(qf, axis=0, keepdims=True)
    dev = qf - mean
    var = jnp.mean(dev * dev, axis=0, keepdims=True)
    out[...] = jnp.maximum(
        dev * lax.rsqrt(var + 1e-5) * gamma[...] + beta[...], 0.0)


@jax.jit
def kernel(ref_bxyz, ref_feat, query_bxyz, e_ref, e_query,
           W_pos, b_pos, W_mlp, b_mlp, bn_gamma, bn_beta):
    f32 = jnp.float32
    table = jnp.concatenate(
        [ref_feat,
         ref_bxyz[:, 1:4],
         jnp.ones((N, 1), f32),
         jnp.zeros((N, D - C - 4), f32)], axis=1)
    eref_w = e_ref.reshape(NW, NCHUNK, K)
    eq_w = e_query.reshape(NW, NCHUNK, K)
    zrows = jnp.zeros((RPS, D), f32)

    partials = _sc_scatter(table, eref_w, eq_w, zrows)
    x = partials.reshape(M, NC * D)

    return pl.pallas_call(
        _dense_body,
        out_shape=jax.ShapeDtypeStruct((M, C), f32),
    )(x, query_bxyz, W_mlp, W_pos, (b_mlp + b_pos).reshape(C, 1),
      bn_gamma.reshape(1, C), bn_beta.reshape(1, C))


# speedup vs baseline: 1.5530x; 1.5530x over previous
import functools

import jax
import jax.numpy as jnp
from jax import lax
from jax.experimental import pallas as pl
from jax.experimental.pallas import tpu as pltpu
from jax.experimental.pallas import tpu_sc as plsc

N = 10000
E = 320000
M = 10000
C = 128
D = 144
NC = 2
NS = 16
NW = NC * NS
EPW = E // NW
K = 125
NCHUNK = EPW // K
RPS = M // NS


def _sc_body(t_hbm, eref_hbm, eq_hbm, z_hbm, out_hbm, idx_v, rows_v, acc_sh,
             sem, isem):
    c = lax.axis_index("c")
    s = lax.axis_index("s")
    wid = c * NS + s
    pltpu.sync_copy(z_hbm, acc_sh.at[pl.ds(s * RPS, RPS)])
    plsc.subcore_barrier()

    def fetch_idx(i, slot, do_async):
        if do_async:
            pltpu.async_copy(eref_hbm.at[wid, i], idx_v.at[slot, 0], isem)
            pltpu.async_copy(eq_hbm.at[wid, i], idx_v.at[slot, 1], isem)
        else:
            pltpu.sync_copy(eref_hbm.at[wid, i], idx_v.at[slot, 0])
            pltpu.sync_copy(eq_hbm.at[wid, i], idx_v.at[slot, 1])

    def wait_idx(i, slot):
        pltpu.make_async_copy(eref_hbm.at[wid, i], idx_v.at[slot, 0],
                              isem).wait()
        pltpu.make_async_copy(eq_hbm.at[wid, i], idx_v.at[slot, 1],
                              isem).wait()

    fetch_idx(0, 0, False)
    fetch_idx(1, 1, True)
    pltpu.async_copy(t_hbm.at[idx_v.at[0, 0]], rows_v.at[0], sem)

    def chunk(i, carry):
        b3 = lax.rem(i, 3)
        b2 = lax.rem(i, 2)
        n3 = lax.rem(i + 1, 3)

        @pl.when(i + 1 < NCHUNK)
        def _fire_gather():
            wait_idx(i + 1, n3)
            pltpu.async_copy(t_hbm.at[idx_v.at[n3, 0]], rows_v.at[1 - b2],
                             sem)

        @pl.when(i + 2 < NCHUNK)
        def _fire_idx():
            fetch_idx(i + 2, lax.rem(i + 2, 3), True)

        pltpu.make_async_copy(t_hbm.at[idx_v.at[b3, 0]], rows_v.at[b2],
                              sem).wait()
        pltpu.sync_copy(rows_v.at[b2], acc_sh.at[idx_v.at[b3, 1]], add=True)
        return carry

    lax.fori_loop(0, NCHUNK, chunk, 0)
    plsc.subcore_barrier()
    pltpu.sync_copy(acc_sh.at[pl.ds(s * RPS, RPS)],
                    out_hbm.at[pl.ds(c * M + s * RPS, RPS)])


def _sc_scatter(table, eref_w, eq_w, zrows):
    mesh = plsc.VectorSubcoreMesh(
        core_axis_name="c", subcore_axis_name="s", num_cores=NC, num_subcores=NS)
    return pl.kernel(
        _sc_body,
        out_type=jax.ShapeDtypeStruct((NC * M, D), jnp.float32),
        mesh=mesh,
        scratch_types=[
            pltpu.VMEM((3, 2, K), jnp.int32),
            pltpu.VMEM((2, K, D), jnp.float32),
            pltpu.VMEM_SHARED((M, D), jnp.float32),
            pltpu.SemaphoreType.DMA,
            pltpu.SemaphoreType.DMA,
        ],
        compiler_params=pltpu.CompilerParams(use_tc_tiling_on_sc=False),
    )(table, eref_w, eq_w, zrows)


def _prep_body(feat, bxyz, out):
    f32 = jnp.float32
    out[...] = jnp.concatenate(
        [feat[...], bxyz[...][:, 1:4], jnp.ones((N, 1), f32),
         jnp.zeros((N, D - C - 4), f32)], axis=1)


def _dense_body(x, qb, wm, wp, bias, gamma, beta, out):
    f32 = jnp.float32
    dg = functools.partial(lax.dot_general, preferred_element_type=f32)
    contract1 = (((1,), (1,)), ((), ()))
    wcomb = jnp.concatenate(
        [wm[...], wp[...], bias[...], jnp.zeros((C, D - C - 4), f32)], axis=1)
    col = lax.broadcasted_iota(jnp.int32, (C, D), 1)
    sel = jnp.where(col == C + 3, 1.0, 0.0)
    wq = jnp.concatenate([jnp.zeros((C, 1), f32), wp[...]], axis=1)

    acc = x[0:M, :] + x[M:2 * M, :]
    counts = dg(acc, sel, contract1)
    numer = dg(acc, wcomb, contract1) - counts * dg(qb[...], wq, contract1)
    qf = numer / jnp.maximum(counts, 1.0)
    mean = jnp.mean(qf, axis=0, keepdims=True)
    dev = qf - mean
    var = jnp.mean(dev * dev, axis=0, keepdims=True)
    out[...] = jnp.maximum(
        dev * lax.rsqrt(var + 1e-5) * gamma[...] + beta[...], 0.0)


@jax.jit
def kernel(ref_bxyz, ref_feat, query_bxyz, e_ref, e_query,
           W_pos, b_pos, W_mlp, b_mlp, bn_gamma, bn_beta):
    f32 = jnp.float32
    table = pl.pallas_call(
        _prep_body,
        out_shape=jax.ShapeDtypeStruct((N, D), f32),
    )(ref_feat, ref_bxyz)
    eref_w = e_ref.reshape(NW, NCHUNK, K)
    eq_w = e_query.reshape(NW, NCHUNK, K)
    zrows = jnp.zeros((RPS, D), f32)

    partials = _sc_scatter(table, eref_w, eq_w, zrows)

    return pl.pallas_call(
        _dense_body,
        out_shape=jax.ShapeDtypeStruct((M, C), f32),
    )(partials, query_bxyz, W_mlp, W_pos, (b_mlp + b_pos).reshape(C, 1),
      bn_gamma.reshape(1, C), bn_beta.reshape(1, C))


# speedup vs baseline: 1.7993x; 1.1586x over previous
import functools

import jax
import jax.numpy as jnp
from jax import lax
from jax.experimental import pallas as pl
from jax.experimental.pallas import tpu as pltpu
from jax.experimental.pallas import tpu_sc as plsc

N = 10000
E = 320000
M = 10000
C = 128
DT = 16
NC = 2
NS = 16
NW = NC * NS
EPW = E // NW
K = 125
NCHUNK = EPW // K
RPS = M // NS


def _sc_body(feat_hbm, tail_hbm, eref_hbm, eq_hbm, zf_hbm, zt_hbm,
             outf_hbm, outt_hbm, idx_v, rowsf_v, rowst_v, accf_sh, acct_sh,
             fsem, tsem, isem):
    c = lax.axis_index("c")
    s = lax.axis_index("s")
    wid = c * NS + s
    pltpu.sync_copy(zf_hbm, accf_sh.at[pl.ds(s * RPS, RPS)])
    pltpu.sync_copy(zt_hbm, acct_sh.at[pl.ds(s * RPS, RPS)])
    plsc.subcore_barrier()

    def fetch_idx(i, slot, do_async):
        if do_async:
            pltpu.async_copy(eref_hbm.at[wid, i], idx_v.at[slot, 0], isem)
            pltpu.async_copy(eq_hbm.at[wid, i], idx_v.at[slot, 1], isem)
        else:
            pltpu.sync_copy(eref_hbm.at[wid, i], idx_v.at[slot, 0])
            pltpu.sync_copy(eq_hbm.at[wid, i], idx_v.at[slot, 1])

    def wait_idx(i, slot):
        pltpu.make_async_copy(eref_hbm.at[wid, i], idx_v.at[slot, 0],
                              isem).wait()
        pltpu.make_async_copy(eq_hbm.at[wid, i], idx_v.at[slot, 1],
                              isem).wait()

    def fire_gathers(slot, buf):
        pltpu.async_copy(feat_hbm.at[idx_v.at[slot, 0]], rowsf_v.at[buf],
                         fsem)
        pltpu.async_copy(tail_hbm.at[idx_v.at[slot, 0]], rowst_v.at[buf],
                         tsem)

    def wait_gathers(slot, buf):
        pltpu.make_async_copy(feat_hbm.at[idx_v.at[slot, 0]],
                              rowsf_v.at[buf], fsem).wait()
        pltpu.make_async_copy(tail_hbm.at[idx_v.at[slot, 0]],
                              rowst_v.at[buf], tsem).wait()

    fetch_idx(0, 0, False)
    fetch_idx(1, 1, True)
    fire_gathers(0, 0)

    def chunk(i, carry):
        b3 = lax.rem(i, 3)
        b2 = lax.rem(i, 2)
        n3 = lax.rem(i + 1, 3)

        @pl.when(i + 1 < NCHUNK)
        def _fire_gather():
            wait_idx(i + 1, n3)
            fire_gathers(n3, 1 - b2)

        @pl.when(i + 2 < NCHUNK)
        def _fire_idx():
            fetch_idx(i + 2, lax.rem(i + 2, 3), True)

        wait_gathers(b3, b2)
        pltpu.sync_copy(rowsf_v.at[b2], accf_sh.at[idx_v.at[b3, 1]],
                        add=True)
        pltpu.sync_copy(rowst_v.at[b2], acct_sh.at[idx_v.at[b3, 1]],
                        add=True)
        return carry

    lax.fori_loop(0, NCHUNK, chunk, 0)
    plsc.subcore_barrier()
    pltpu.sync_copy(accf_sh.at[pl.ds(s * RPS, RPS)],
                    outf_hbm.at[pl.ds(c * M + s * RPS, RPS)])
    pltpu.sync_copy(acct_sh.at[pl.ds(s * RPS, RPS)],
                    outt_hbm.at[pl.ds(c * M + s * RPS, RPS)])


def _sc_scatter(feat, tail, eref_w, eq_w, zf, zt):
    mesh = plsc.VectorSubcoreMesh(
        core_axis_name="c", subcore_axis_name="s", num_cores=NC, num_subcores=NS)
    return pl.kernel(
        _sc_body,
        out_type=(jax.ShapeDtypeStruct((NC * M, C), jnp.float32),
                  jax.ShapeDtypeStruct((NC * M, DT), jnp.float32)),
        mesh=mesh,
        scratch_types=[
            pltpu.VMEM((3, 2, K), jnp.int32),
            pltpu.VMEM((2, K, C), jnp.float32),
            pltpu.VMEM((2, K, DT), jnp.float32),
            pltpu.VMEM_SHARED((M, C), jnp.float32),
            pltpu.VMEM_SHARED((M, DT), jnp.float32),
            pltpu.SemaphoreType.DMA,
            pltpu.SemaphoreType.DMA,
            pltpu.SemaphoreType.DMA,
        ],
        compiler_params=pltpu.CompilerParams(use_tc_tiling_on_sc=False),
    )(feat, tail, eref_w, eq_w, zf, zt)


def _dense_body(xf, xt, qb, wm, wp, bias, gamma, beta, out):
    f32 = jnp.float32
    dg = functools.partial(lax.dot_general, preferred_element_type=f32)
    contract1 = (((1,), (1,)), ((), ()))
    wt = jnp.concatenate(
        [wp[...], bias[...], jnp.zeros((C, DT - 4), f32)], axis=1)
    col = lax.broadcasted_iota(jnp.int32, (C, DT), 1)
    sel = jnp.where(col == 3, 1.0, 0.0)
    wq = jnp.concatenate([jnp.zeros((C, 1), f32), wp[...]], axis=1)

    accf = xf[0:M, :] + xf[M:2 * M, :]
    acct = xt[0:M, :] + xt[M:2 * M, :]
    counts = dg(acct, sel, contract1)
    numer = (dg(accf, wm[...], contract1) + dg(acct, wt, contract1)
             - counts * dg(qb[...], wq, contract1))
    qf = numer / jnp.maximum(counts, 1.0)
    mean = jnp.mean(qf, axis=0, keepdims=True)
    dev = qf - mean
    var = jnp.mean(dev * dev, axis=0, keepdims=True)
    out[...] = jnp.maximum(
        dev * lax.rsqrt(var + 1e-5) * gamma[...] + beta[...], 0.0)


@jax.jit
def kernel(ref_bxyz, ref_feat, query_bxyz, e_ref, e_query,
           W_pos, b_pos, W_mlp, b_mlp, bn_gamma, bn_beta):
    f32 = jnp.float32
    tail = jnp.concatenate(
        [ref_bxyz[:, 1:4], jnp.ones((N, 1), f32),
         jnp.zeros((N, DT - 4), f32)], axis=1)
    eref_w = e_ref.reshape(NW, NCHUNK, K)
    eq_w = e_query.reshape(NW, NCHUNK, K)
    zf = jnp.zeros((RPS, C), f32)
    zt = jnp.zeros((RPS, DT), f32)

    xf, xt = _sc_scatter(ref_feat, tail, eref_w, eq_w, zf, zt)

    return pl.pallas_call(
        _dense_body,
        out_shape=jax.ShapeDtypeStruct((M, C), f32),
    )(xf, xt, query_bxyz, W_mlp, W_pos, (b_mlp + b_pos).reshape(C, 1),
      bn_gamma.reshape(1, C), bn_beta.reshape(1, C))


# speedup vs baseline: 1.8060x; 1.0037x over previous
import functools

import jax
import jax.numpy as jnp
from jax import lax
from jax.experimental import pallas as pl
from jax.experimental.pallas import tpu as pltpu
from jax.experimental.pallas import tpu_sc as plsc

N = 10000
E = 320000
M = 10000
C = 128
DT = 16
NC = 2
NS = 16
NW = NC * NS
EPW = E // NW
K = 125
NCHUNK = EPW // K
RPS = M // NS


def _sc_body(feat_hbm, tail_hbm, eref_hbm, eq_hbm, zf_hbm, zt_hbm,
             outf_hbm, outt_hbm, idx_v, rowsf_v, rowst_v, accf_sh, acct_sh,
             fsem, tsem, isem, ssem):
    c = lax.axis_index("c")
    s = lax.axis_index("s")
    wid = c * NS + s
    pltpu.sync_copy(zf_hbm, accf_sh.at[pl.ds(s * RPS, RPS)])
    pltpu.sync_copy(zt_hbm, acct_sh.at[pl.ds(s * RPS, RPS)])
    plsc.subcore_barrier()

    def fetch_idx(i, slot, do_async):
        if do_async:
            pltpu.async_copy(eref_hbm.at[wid, i], idx_v.at[slot, 0], isem)
            pltpu.async_copy(eq_hbm.at[wid, i], idx_v.at[slot, 1], isem)
        else:
            pltpu.sync_copy(eref_hbm.at[wid, i], idx_v.at[slot, 0])
            pltpu.sync_copy(eq_hbm.at[wid, i], idx_v.at[slot, 1])

    def wait_idx(i, slot):
        pltpu.make_async_copy(eref_hbm.at[wid, i], idx_v.at[slot, 0],
                              isem).wait()
        pltpu.make_async_copy(eq_hbm.at[wid, i], idx_v.at[slot, 1],
                              isem).wait()

    def fire_gathers(slot, buf):
        pltpu.async_copy(feat_hbm.at[idx_v.at[slot, 0]], rowsf_v.at[buf],
                         fsem)
        pltpu.async_copy(tail_hbm.at[idx_v.at[slot, 0]], rowst_v.at[buf],
                         tsem)

    def wait_gathers(slot, buf):
        pltpu.make_async_copy(feat_hbm.at[idx_v.at[slot, 0]],
                              rowsf_v.at[buf], fsem).wait()
        pltpu.make_async_copy(tail_hbm.at[idx_v.at[slot, 0]],
                              rowst_v.at[buf], tsem).wait()

    def fire_scatters(slot, buf):
        pltpu.async_copy(rowsf_v.at[buf], accf_sh.at[idx_v.at[slot, 1]],
                         ssem, add=True)
        pltpu.async_copy(rowst_v.at[buf], acct_sh.at[idx_v.at[slot, 1]],
                         ssem, add=True)

    def wait_scatters(slot, buf):
        pltpu.make_async_copy(rowsf_v.at[buf],
                              accf_sh.at[idx_v.at[slot, 1]], ssem).wait()
        pltpu.make_async_copy(rowst_v.at[buf],
                              acct_sh.at[idx_v.at[slot, 1]], ssem).wait()

    fetch_idx(0, 0, False)
    fetch_idx(1, 1, True)
    fire_gathers(0, 0)

    def chunk(i, carry):
        b3 = lax.rem(i, 3)
        b2 = lax.rem(i, 2)
        n3 = lax.rem(i + 1, 3)

        @pl.when(i + 1 < NCHUNK)
        def _fire_gather():
            wait_idx(i + 1, n3)

            @pl.when(i >= 1)
            def _free_buf():
                wait_scatters(lax.rem(i - 1, 3), 1 - b2)

            fire_gathers(n3, 1 - b2)

        @pl.when(i + 2 < NCHUNK)
        def _fire_idx():
            fetch_idx(i + 2, lax.rem(i + 2, 3), True)

        wait_gathers(b3, b2)
        fire_scatters(b3, b2)
        return carry

    lax.fori_loop(0, NCHUNK, chunk, 0)
    wait_scatters(lax.rem(NCHUNK - 2, 3), lax.rem(NCHUNK - 2, 2))
    wait_scatters(lax.rem(NCHUNK - 1, 3), lax.rem(NCHUNK - 1, 2))
    plsc.subcore_barrier()
    pltpu.sync_copy(accf_sh.at[pl.ds(s * RPS, RPS)],
                    outf_hbm.at[pl.ds(c * M + s * RPS, RPS)])
    pltpu.sync_copy(acct_sh.at[pl.ds(s * RPS, RPS)],
                    outt_hbm.at[pl.ds(c * M + s * RPS, RPS)])


def _sc_scatter(feat, tail, eref_w, eq_w, zf, zt):
    mesh = plsc.VectorSubcoreMesh(
        core_axis_name="c", subcore_axis_name="s", num_cores=NC, num_subcores=NS)
    return pl.kernel(
        _sc_body,
        out_type=(jax.ShapeDtypeStruct((NC * M, C), jnp.float32),
                  jax.ShapeDtypeStruct((NC * M, DT), jnp.float32)),
        mesh=mesh,
        scratch_types=[
            pltpu.VMEM((3, 2, K), jnp.int32),
            pltpu.VMEM((2, K, C), jnp.float32),
            pltpu.VMEM((2, K, DT), jnp.float32),
            pltpu.VMEM_SHARED((M, C), jnp.float32),
            pltpu.VMEM_SHARED((M, DT), jnp.float32),
            pltpu.SemaphoreType.DMA,
            pltpu.SemaphoreType.DMA,
            pltpu.SemaphoreType.DMA,
            pltpu.SemaphoreType.DMA,
        ],
        compiler_params=pltpu.CompilerParams(use_tc_tiling_on_sc=False),
    )(feat, tail, eref_w, eq_w, zf, zt)


def _dense_body(xf, xt, qb, wm, wp, bias, gamma, beta, out):
    f32 = jnp.float32
    dg = functools.partial(lax.dot_general, preferred_element_type=f32)
    contract1 = (((1,), (1,)), ((), ()))
    wt = jnp.concatenate(
        [wp[...], bias[...], jnp.zeros((C, DT - 4), f32)], axis=1)
    col = lax.broadcasted_iota(jnp.int32, (C, DT), 1)
    sel = jnp.where(col == 3, 1.0, 0.0)
    wq = jnp.concatenate([jnp.zeros((C, 1), f32), wp[...]], axis=1)

    accf = xf[0:M, :] + xf[M:2 * M, :]
    acct = xt[0:M, :] + xt[M:2 * M, :]
    counts = dg(acct, sel, contract1)
    numer = (dg(accf, wm[...], contract1) + dg(acct, wt, contract1)
             - counts * dg(qb[...], wq, contract1))
    qf = numer / jnp.maximum(counts, 1.0)
    mean = jnp.mean(qf, axis=0, keepdims=True)
    dev = qf - mean
    var = jnp.mean(dev * dev, axis=0, keepdims=True)
    out[...] = jnp.maximum(
        dev * lax.rsqrt(var + 1e-5) * gamma[...] + beta[...], 0.0)


@jax.jit
def kernel(ref_bxyz, ref_feat, query_bxyz, e_ref, e_query,
           W_pos, b_pos, W_mlp, b_mlp, bn_gamma, bn_beta):
    f32 = jnp.float32
    tail = jnp.concatenate(
        [ref_bxyz[:, 1:4], jnp.ones((N, 1), f32),
         jnp.zeros((N, DT - 4), f32)], axis=1)
    eref_w = e_ref.reshape(NW, NCHUNK, K)
    eq_w = e_query.reshape(NW, NCHUNK, K)
    zf = jnp.zeros((RPS, C), f32)
    zt = jnp.zeros((RPS, DT), f32)

    xf, xt = _sc_scatter(ref_feat, tail, eref_w, eq_w, zf, zt)

    return pl.pallas_call(
        _dense_body,
        out_shape=jax.ShapeDtypeStruct((M, C), f32),
    )(xf, xt, query_bxyz, W_mlp, W_pos, (b_mlp + b_pos).reshape(C, 1),
      bn_gamma.reshape(1, C), bn_beta.reshape(1, C))
